# Initial kernel scaffold; baseline (speedup 1.0000x reference)
import jax
import jax.numpy as jnp
from jax.experimental import pallas as pl


def kernel(xyz1, xyz2, points1, points2, point_lens, embedding_lens, point_mask, W, gamma, beta):
    raise NotImplementedError("write your pallas kernel here")


# speedup vs baseline: 7.9123x; 7.9123x over previous
import jax
import jax.numpy as jnp
from jax.experimental import pallas as pl

_K = 8
_TN = 256


def _phase1(xyz1_ref, xyz2t_ref, p1_ref, p2_ref, qm_ref, km_ref, w1t_ref,
            w2t_ref, y_ref, idx_ref, stats_ref):
    b = pl.program_id(0)
    i = pl.program_id(1)
    q = xyz1_ref[0]
    k8s = xyz2t_ref[0]
    qn = jnp.sum(q * q, axis=1, keepdims=True)
    kn = jnp.sum(k8s * k8s, axis=0, keepdims=True)
    qk = jnp.dot(q, k8s, preferred_element_type=jnp.float32)
    d2 = qn + kn - 2.0 * qk
    km = km_ref[0]
    d2 = jnp.where(km > 0.0, d2, jnp.inf)
    vq = qm_ref[0] > 0.0
    TN, S = d2.shape
    iota = jax.lax.broadcasted_iota(jnp.int32, (TN, S), 1)
    eps = jnp.float32(jnp.finfo(jnp.float32).eps)
    acc = jnp.zeros((TN, S), jnp.float32)
    norm = jnp.zeros((TN, 1), jnp.float32)
    cols = []
    d2m = d2
    for _ in range(_K):
        m = jnp.min(d2m, axis=1, keepdims=True)
        am = jnp.min(jnp.where(d2m == m, iota, S), axis=1, keepdims=True)
        d2m = jnp.where(iota == am, jnp.inf, d2m)
        dk = jnp.where(vq, m, 0.0)
        ik = jnp.where(vq, am, 0)
        rk = 1.0 / (dk + eps)
        norm = norm + rk
        acc = acc + jnp.where(iota == ik, rk, 0.0)
        cols.append(ik)
    idx_ref[0] = jnp.concatenate(cols, axis=1)
    wmat = acc / norm
    interp = jnp.dot(wmat, p2_ref[0], preferred_element_type=jnp.float32)
    y = (jnp.dot(p1_ref[0], w1t_ref[...], preferred_element_type=jnp.float32)
         + jnp.dot(interp, w2t_ref[...], preferred_element_type=jnp.float32))
    y_ref[0] = y
    vqf = qm_ref[0]
    ym = y * vqf
    s = jnp.sum(ym, axis=0, keepdims=True)
    ss = jnp.sum(y * ym, axis=0, keepdims=True)
    st = jnp.concatenate(
        [s, ss, jnp.zeros((6, s.shape[1]), jnp.float32)], axis=0)

    first = jnp.logical_and(b == 0, i == 0)

    @pl.when(first)
    def _():
        stats_ref[...] = st

    @pl.when(jnp.logical_not(first))
    def _():
        stats_ref[...] = stats_ref[...] + st


def _phase2(y_ref, sb_ref, o_ref):
    o_ref[0] = y_ref[0] * sb_ref[0:1, :] + sb_ref[1:2, :]


def kernel(xyz1, xyz2, points1, points2, point_lens, embedding_lens,
           point_mask, W, gamma, beta):
    B, N, _ = xyz1.shape
    S = xyz2.shape[1]
    D1 = points1.shape[2]
    D2 = points2.shape[2]
    C = W.shape[0]
    f32 = jnp.float32

    q8 = jnp.concatenate(
        [xyz1[..., :3], jnp.zeros((B, N, 5), f32)], axis=-1)
    k8 = jnp.concatenate(
        [xyz2, jnp.zeros((B, S, 5), f32)], axis=-1)
    xyz2t = jnp.transpose(k8, (0, 2, 1))
    p1p = jnp.concatenate(
        [points1, jnp.zeros((B, N, 8 - D1), f32)], axis=-1)
    qm = point_mask.astype(f32).reshape(B, N, 1)
    km = (jnp.arange(S)[None, :] < embedding_lens[:, None]
          ).astype(f32).reshape(B, 1, S)
    w1t = jnp.concatenate(
        [W[:, :D1].T, jnp.zeros((8 - D1, C), f32)], axis=0)
    w2t = W[:, D1:].T

    y, idx, stats = pl.pallas_call(
        _phase1,
        grid=(B, N // _TN),
        in_specs=[
            pl.BlockSpec((1, _TN, 8), lambda b, i: (b, i, 0)),
            pl.BlockSpec((1, 8, S), lambda b, i: (b, 0, 0)),
            pl.BlockSpec((1, _TN, 8), lambda b, i: (b, i, 0)),
            pl.BlockSpec((1, S, D2), lambda b, i: (b, 0, 0)),
            pl.BlockSpec((1, _TN, 1), lambda b, i: (b, i, 0)),
            pl.BlockSpec((1, 1, S), lambda b, i: (b, 0, 0)),
            pl.BlockSpec((8, C), lambda b, i: (0, 0)),
            pl.BlockSpec((D2, C), lambda b, i: (0, 0)),
        ],
        out_specs=[
            pl.BlockSpec((1, _TN, C), lambda b, i: (b, i, 0)),
            pl.BlockSpec((1, _TN, _K), lambda b, i: (b, i, 0)),
            pl.BlockSpec((8, C), lambda b, i: (0, 0)),
        ],
        out_shape=[
            jax.ShapeDtypeStruct((B, N, C), f32),
            jax.ShapeDtypeStruct((B, N, _K), jnp.int32),
            jax.ShapeDtypeStruct((8, C), f32),
        ],
    )(q8, xyz2t, p1p, points2, qm, km, w1t, w2t)

    cnt = jnp.sum(point_mask.astype(f32))
    mean = stats[0] / cnt
    var = stats[1] / cnt - mean * mean
    scale = gamma / jnp.sqrt(var + 1e-5)
    bias = beta - mean * scale
    sb = jnp.stack([scale, bias], axis=0)

    tn2 = 512
    out = pl.pallas_call(
        _phase2,
        grid=(B, N // tn2),
        in_specs=[
            pl.BlockSpec((1, tn2, C), lambda b, i: (b, i, 0)),
            pl.BlockSpec((2, C), lambda b, i: (0, 0)),
        ],
        out_specs=pl.BlockSpec((1, tn2, C), lambda b, i: (b, i, 0)),
        out_shape=jax.ShapeDtypeStruct((B, N, C), f32),
    )(y, sb)
    return out, idx


# speedup vs baseline: 14.6295x; 1.8489x over previous
import jax
import jax.numpy as jnp
from jax.experimental import pallas as pl

_K = 8
_TN = 256


def _phase1(qa_ref, ka_ref, p1_ref, p2_ref, qm_ref, km_ref, w1t_ref,
            w2t_ref, y_ref, idx_ref, stats_ref):
    b = pl.program_id(0)
    i = pl.program_id(1)
    q = qa_ref[0]
    k8s = ka_ref[0]
    qn = jnp.sum(q * q, axis=1, keepdims=True)
    kn = jnp.sum(k8s * k8s, axis=0, keepdims=True)
    qk = jnp.dot(q, k8s, preferred_element_type=jnp.float32)
    d2 = qn + kn - 2.0 * qk
    km = km_ref[0]
    d2 = jnp.where(km > 0.0, d2, jnp.inf)
    vq = qm_ref[0] > 0.0
    TN, S = d2.shape
    iota = jax.lax.broadcasted_iota(jnp.int32, (TN, S), 1)
    eps = jnp.float32(jnp.finfo(jnp.float32).eps)
    norm = jnp.zeros((TN, 1), jnp.float32)
    cols = []
    d2m = d2
    for _ in range(_K):
        m = jnp.min(d2m, axis=1, keepdims=True)
        eq = d2m == m
        am = jnp.min(jnp.where(eq, iota, S), axis=1, keepdims=True)
        d2m = jnp.where(iota == am, jnp.inf, d2m)
        dk = jnp.where(vq, m, 0.0)
        ik = jnp.where(vq, am, 0)
        rk = 1.0 / (dk + eps)
        norm = norm + rk
        cols.append(ik)
    idx_ref[0] = jnp.concatenate(cols, axis=1)
    sel = jnp.logical_and(d2m == jnp.inf, d2 != jnp.inf)
    rmat = jnp.where(sel, 1.0 / (d2 + eps), 0.0)
    wmat = jnp.where(vq, rmat / norm,
                     jnp.where(iota == 0, 1.0, 0.0))
    interp = jnp.dot(wmat, p2_ref[0], preferred_element_type=jnp.float32)
    y = (jnp.dot(p1_ref[0], w1t_ref[...], preferred_element_type=jnp.float32)
         + jnp.dot(interp, w2t_ref[...], preferred_element_type=jnp.float32))
    y_ref[0] = y
    vqf = qm_ref[0]
    ym = y * vqf
    s = jnp.sum(ym, axis=0, keepdims=True)
    ss = jnp.sum(y * ym, axis=0, keepdims=True)
    st = jnp.concatenate(
        [s, ss, jnp.zeros((6, s.shape[1]), jnp.float32)], axis=0)

    first = jnp.logical_and(b == 0, i == 0)

    @pl.when(first)
    def _():
        stats_ref[...] = st

    @pl.when(jnp.logical_not(first))
    def _():
        stats_ref[...] = stats_ref[...] + st


def _phase2(y_ref, sb_ref, o_ref):
    o_ref[0] = y_ref[0] * sb_ref[0:1, :] + sb_ref[1:2, :]


def kernel(xyz1, xyz2, points1, points2, point_lens, embedding_lens,
           point_mask, W, gamma, beta):
    B, N, _ = xyz1.shape
    S = xyz2.shape[1]
    D1 = points1.shape[2]
    D2 = points2.shape[2]
    C = W.shape[0]
    f32 = jnp.float32

    qa = jnp.concatenate(
        [xyz1[..., :3], jnp.zeros((B, N, 5), f32)], axis=-1)
    ka = jnp.transpose(
        jnp.concatenate([xyz2, jnp.zeros((B, S, 5), f32)], axis=-1),
        (0, 2, 1))
    p1p = jnp.concatenate(
        [points1, jnp.zeros((B, N, 8 - D1), f32)], axis=-1)
    qm = point_mask.astype(f32).reshape(B, N, 1)
    km = (jnp.arange(S)[None, :] < embedding_lens[:, None]
          ).astype(f32).reshape(B, 1, S)
    w1t = jnp.concatenate(
        [W[:, :D1].T, jnp.zeros((8 - D1, C), f32)], axis=0)
    w2t = W[:, D1:].T

    y, idx, stats = pl.pallas_call(
        _phase1,
        grid=(B, N // _TN),
        in_specs=[
            pl.BlockSpec((1, _TN, 8), lambda b, i: (b, i, 0)),
            pl.BlockSpec((1, 8, S), lambda b, i: (b, 0, 0)),
            pl.BlockSpec((1, _TN, 8), lambda b, i: (b, i, 0)),
            pl.BlockSpec((1, S, D2), lambda b, i: (b, 0, 0)),
            pl.BlockSpec((1, _TN, 1), lambda b, i: (b, i, 0)),
            pl.BlockSpec((1, 1, S), lambda b, i: (b, 0, 0)),
            pl.BlockSpec((8, C), lambda b, i: (0, 0)),
            pl.BlockSpec((D2, C), lambda b, i: (0, 0)),
        ],
        out_specs=[
            pl.BlockSpec((1, _TN, C), lambda b, i: (b, i, 0)),
            pl.BlockSpec((1, _TN, _K), lambda b, i: (b, i, 0)),
            pl.BlockSpec((8, C), lambda b, i: (0, 0)),
        ],
        out_shape=[
            jax.ShapeDtypeStruct((B, N, C), f32),
            jax.ShapeDtypeStruct((B, N, _K), jnp.int32),
            jax.ShapeDtypeStruct((8, C), f32),
        ],
    )(qa, ka, p1p, points2, qm, km, w1t, w2t)

    cnt = jnp.sum(point_mask.astype(f32))
    mean = stats[0] / cnt
    var = stats[1] / cnt - mean * mean
    scale = gamma / jnp.sqrt(var + 1e-5)
    bias = beta - mean * scale
    sb = jnp.stack([scale, bias], axis=0)

    tn2 = 512
    out = pl.pallas_call(
        _phase2,
        grid=(B, N // tn2),
        in_specs=[
            pl.BlockSpec((1, tn2, C), lambda b, i: (b, i, 0)),
            pl.BlockSpec((2, C), lambda b, i: (0, 0)),
        ],
        out_specs=pl.BlockSpec((1, tn2, C), lambda b, i: (b, i, 0)),
        out_shape=jax.ShapeDtypeStruct((B, N, C), f32),
    )(y, sb)
    return out, idx


# speedup vs baseline: 17.2333x; 1.1780x over previous
import jax
import jax.numpy as jnp
from jax.experimental import pallas as pl

_K = 8
_TN = 256


def _phase1(qa_ref, ka_ref, p1_ref, p2_ref, qm_ref, km_ref, iotaf_ref,
            w1t_ref, w2t_ref, y_ref, idx_ref, stats_ref):
    b = pl.program_id(0)
    i = pl.program_id(1)
    q = qa_ref[0]
    k8s = ka_ref[0]
    qn = jnp.sum(q * q, axis=1, keepdims=True)
    kn = jnp.sum(k8s * k8s, axis=0, keepdims=True)
    qk = jnp.dot(q, k8s, preferred_element_type=jnp.float32)
    d2 = qn + kn - 2.0 * qk
    km = km_ref[0]
    d2 = jnp.where(km > 0.0, d2, jnp.inf)
    vq = qm_ref[0] > 0.0
    TN, S = d2.shape
    iotaf = iotaf_ref[0]
    eps = jnp.float32(jnp.finfo(jnp.float32).eps)
    norm = jnp.zeros((TN, 1), jnp.float32)
    cols = []
    d2m = d2
    for _ in range(_K):
        m = jnp.min(d2m, axis=1, keepdims=True)
        eq = d2m == m
        amf = jnp.min(jnp.where(eq, iotaf, jnp.float32(S)), axis=1,
                      keepdims=True)
        d2m = jnp.where(iotaf == amf, jnp.inf, d2m)
        dk = jnp.where(vq, m, 0.0)
        ikf = jnp.where(vq, amf, 0.0)
        rk = 1.0 / (dk + eps)
        norm = norm + rk
        cols.append(ikf)
    idx_ref[0] = jnp.concatenate(cols, axis=1)
    sel = jnp.logical_and(d2m == jnp.inf, d2 != jnp.inf)
    rmat = jnp.where(sel, 1.0 / (d2 + eps), 0.0)
    wmat = jnp.where(vq, rmat / norm,
                     jnp.where(iotaf == 0.0, 1.0, 0.0))
    interp = jnp.dot(wmat, p2_ref[0], preferred_element_type=jnp.float32)
    y = (jnp.dot(p1_ref[0], w1t_ref[...], preferred_element_type=jnp.float32)
         + jnp.dot(interp, w2t_ref[...], preferred_element_type=jnp.float32))
    y_ref[0] = y
    vqf = qm_ref[0]
    ym = y * vqf
    s = jnp.sum(ym, axis=0, keepdims=True)
    ss = jnp.sum(y * ym, axis=0, keepdims=True)
    st = jnp.concatenate(
        [s, ss, jnp.zeros((6, s.shape[1]), jnp.float32)], axis=0)

    first = jnp.logical_and(b == 0, i == 0)

    @pl.when(first)
    def _():
        stats_ref[...] = st

    @pl.when(jnp.logical_not(first))
    def _():
        stats_ref[...] = stats_ref[...] + st


def _phase2(y_ref, sb_ref, o_ref):
    o_ref[0] = y_ref[0] * sb_ref[0:1, :] + sb_ref[1:2, :]


def kernel(xyz1, xyz2, points1, points2, point_lens, embedding_lens,
           point_mask, W, gamma, beta):
    B, N, _ = xyz1.shape
    S = xyz2.shape[1]
    D1 = points1.shape[2]
    D2 = points2.shape[2]
    C = W.shape[0]
    f32 = jnp.float32

    qa = jnp.concatenate(
        [xyz1[..., :3], jnp.zeros((B, N, 5), f32)], axis=-1)
    ka = jnp.transpose(
        jnp.concatenate([xyz2, jnp.zeros((B, S, 5), f32)], axis=-1),
        (0, 2, 1))
    p1p = jnp.concatenate(
        [points1, jnp.zeros((B, N, 8 - D1), f32)], axis=-1)
    qm = point_mask.astype(f32).reshape(B, N, 1)
    km = (jnp.arange(S)[None, :] < embedding_lens[:, None]
          ).astype(f32).reshape(B, 1, S)
    iotaf_in = jnp.arange(S, dtype=f32).reshape(1, 1, S)
    w1t = jnp.concatenate(
        [W[:, :D1].T, jnp.zeros((8 - D1, C), f32)], axis=0)
    w2t = W[:, D1:].T

    y, idx, stats = pl.pallas_call(
        _phase1,
        grid=(B, N // _TN),
        in_specs=[
            pl.BlockSpec((1, _TN, 8), lambda b, i: (b, i, 0)),
            pl.BlockSpec((1, 8, S), lambda b, i: (b, 0, 0)),
            pl.BlockSpec((1, _TN, 8), lambda b, i: (b, i, 0)),
            pl.BlockSpec((1, S, D2), lambda b, i: (b, 0, 0)),
            pl.BlockSpec((1, _TN, 1), lambda b, i: (b, i, 0)),
            pl.BlockSpec((1, 1, S), lambda b, i: (b, 0, 0)),
            pl.BlockSpec((1, 1, S), lambda b, i: (0, 0, 0)),
            pl.BlockSpec((8, C), lambda b, i: (0, 0)),
            pl.BlockSpec((D2, C), lambda b, i: (0, 0)),
        ],
        out_specs=[
            pl.BlockSpec((1, _TN, C), lambda b, i: (b, i, 0)),
            pl.BlockSpec((1, _TN, _K), lambda b, i: (b, i, 0)),
            pl.BlockSpec((8, C), lambda b, i: (0, 0)),
        ],
        out_shape=[
            jax.ShapeDtypeStruct((B, N, C), f32),
            jax.ShapeDtypeStruct((B, N, _K), f32),
            jax.ShapeDtypeStruct((8, C), f32),
        ],
    )(qa, ka, p1p, points2, qm, km, iotaf_in, w1t, w2t)
    idx = idx.astype(jnp.int32)

    cnt = jnp.sum(point_mask.astype(f32))
    mean = stats[0] / cnt
    var = stats[1] / cnt - mean * mean
    scale = gamma / jnp.sqrt(var + 1e-5)
    bias = beta - mean * scale
    sb = jnp.stack([scale, bias], axis=0)

    tn2 = 512
    out = pl.pallas_call(
        _phase2,
        grid=(B, N // tn2),
        in_specs=[
            pl.BlockSpec((1, tn2, C), lambda b, i: (b, i, 0)),
            pl.BlockSpec((2, C), lambda b, i: (0, 0)),
        ],
        out_specs=pl.BlockSpec((1, tn2, C), lambda b, i: (b, i, 0)),
        out_shape=jax.ShapeDtypeStruct((B, N, C), f32),
    )(y, sb)
    return out, idx
